# baseline (speedup 1.0000x reference)
import functools

import jax
import jax.numpy as jnp
from jax import lax
from jax.experimental import pallas as pl
from jax.experimental.pallas import tpu as pltpu
from jax.experimental.pallas import tpu_sc as plsc

N = 10000
E = 160000
D_IN = 256
D1 = 100
D1P = 112
D2 = 64
D3 = 32

NC = 2
NS = 16
NT = NC * NS
NPAD = 10240
SL = NPAD // NS
CHE = 128
NCHT = 40
EPT = NCHT * CHE
EPAD = NT * EPT

BR = 1024
GRID = NPAD // BR

_mesh = plsc.VectorSubcoreMesh(
    core_axis_name="c", subcore_axis_name="s", num_cores=NC, num_subcores=NS)
_sc_params = pltpu.CompilerParams(
    needs_layout_passes=False, use_tc_tiling_on_sc=False)


def _wid():
    return lax.axis_index("c") * NS + lax.axis_index("s")


def _zero_vmem_1d(ref, n):
    z = jnp.zeros((16,), jnp.float32)

    def body(i, _):
        ref[pl.ds(i * 16, 16)] = z
        return _

    lax.fori_loop(0, n // 16, body, None)


def _combine_and_store(hist, shared, red, outb, out_hbm):
    sid = lax.axis_index("s")
    cid = lax.axis_index("c")
    pltpu.sync_copy(hist, shared.at[sid])
    plsc.subcore_barrier()
    for k in range(NS):
        pltpu.sync_copy(shared.at[k, pl.ds(sid * SL, SL)], red.at[k])

    def body(j, _):
        sl = pl.ds(j * 16, 16)
        acc = red[0, sl]
        for k in range(1, NS):
            acc = acc + red[k, sl]
        outb[sl] = acc
        return _

    lax.fori_loop(0, SL // 16, body, None)
    pltpu.sync_copy(outb, out_hbm.at[cid, pl.ds(sid * SL, SL)])


@functools.partial(
    pl.kernel,
    out_type=jax.ShapeDtypeStruct((NC, NPAD), jnp.float32),
    mesh=_mesh,
    compiler_params=_sc_params,
    scratch_types=[
        pltpu.VMEM((NPAD,), jnp.float32),
        pltpu.VMEM((NCHT, CHE), jnp.int32),
        pltpu.VMEM_SHARED((NS, NPAD), jnp.float32),
        pltpu.VMEM((NS, SL), jnp.float32),
        pltpu.VMEM((SL,), jnp.float32),
    ],
)
def _deg_call(dstc_hbm, out_hbm, hist, dif, shared, red, outb):
    wid = _wid()
    pltpu.sync_copy(dstc_hbm.at[wid], dif)
    _zero_vmem_1d(hist, NPAD)
    ones = jnp.ones((16,), jnp.float32)

    def body(i, _):
        def inner(j, _2):
            idx = dif[i, pl.ds(j * 16, 16)]
            plsc.addupdate_scatter(hist, [idx], ones)
            return _2

        lax.fori_loop(0, CHE // 16, inner, None)
        return _

    lax.fori_loop(0, NCHT, body, None)
    _combine_and_store(hist, shared, red, outb, out_hbm)


def _ring_body(hp_hbm, sidx, didx, rows0, rows1, gs0, gs1, acc, gpart=None):
    pltpu.async_copy(hp_hbm.at[sidx.at[0]], rows0, gs0)

    def body(i, _):
        c0 = 2 * i
        c1 = c0 + 1
        pltpu.async_copy(hp_hbm.at[sidx.at[c1]], rows1, gs1)
        if gpart is not None:
            gpart(c0)
        pltpu.make_async_copy(hp_hbm.at[sidx.at[c0]], rows0, gs0).wait()
        pltpu.sync_copy(rows0, acc.at[didx.at[c0]], add=True)

        @pl.when(i < NCHT // 2 - 1)
        def _nx():
            pltpu.async_copy(hp_hbm.at[sidx.at[c0 + 2]], rows0, gs0)

        if gpart is not None:
            gpart(c1)
        pltpu.make_async_copy(hp_hbm.at[sidx.at[c1]], rows1, gs1).wait()
        pltpu.sync_copy(rows1, acc.at[didx.at[c1]], add=True)
        return _

    lax.fori_loop(0, NCHT // 2, body, None)


@functools.partial(
    pl.kernel,
    out_type=jax.ShapeDtypeStruct((NC, NPAD, D1P), jnp.float32),
    mesh=_mesh,
    compiler_params=_sc_params,
    scratch_types=[
        pltpu.VMEM((NCHT, CHE), jnp.int32),
        pltpu.VMEM((NCHT, CHE), jnp.int32),
        pltpu.VMEM((CHE, D1P), jnp.float32),
        pltpu.VMEM((CHE, D1P), jnp.float32),
        pltpu.SemaphoreType.DMA,
        pltpu.SemaphoreType.DMA,
        pltpu.VMEM_SHARED((NPAD, D1P), jnp.float32),
    ],
)
def _agg1_call(hp_hbm, srcc_hbm, dstc_hbm, zer_hbm, s_hbm, sidx, didx,
               rows0, rows1, gs0, gs1, acc):
    cid = lax.axis_index("c")
    sid = lax.axis_index("s")
    wid = cid * NS + sid
    pltpu.sync_copy(zer_hbm, acc.at[pl.ds(sid * SL, SL)])
    pltpu.sync_copy(srcc_hbm.at[wid], sidx)
    pltpu.sync_copy(dstc_hbm.at[wid], didx)
    plsc.subcore_barrier()
    _ring_body(hp_hbm, sidx, didx, rows0, rows1, gs0, gs1, acc)
    plsc.subcore_barrier()
    pltpu.sync_copy(acc.at[pl.ds(sid * SL, SL)],
                    s_hbm.at[cid, pl.ds(sid * SL, SL)])


@functools.partial(
    pl.kernel,
    out_type=[jax.ShapeDtypeStruct((NC, NPAD, D2), jnp.float32),
              jax.ShapeDtypeStruct((NC, NPAD), jnp.float32)],
    mesh=_mesh,
    compiler_params=_sc_params,
    scratch_types=[
        pltpu.VMEM((NCHT, CHE), jnp.int32),
        pltpu.VMEM((NCHT, CHE), jnp.int32),
        pltpu.VMEM((CHE, D2), jnp.float32),
        pltpu.VMEM((CHE, D2), jnp.float32),
        pltpu.SemaphoreType.DMA,
        pltpu.SemaphoreType.DMA,
        pltpu.VMEM((NPAD,), jnp.float32),
        pltpu.VMEM((NPAD,), jnp.float32),
        pltpu.VMEM_SHARED((NPAD, D2), jnp.float32),
        pltpu.VMEM_SHARED((NS, NPAD), jnp.float32),
        pltpu.VMEM((NS, SL), jnp.float32),
        pltpu.VMEM((SL,), jnp.float32),
    ],
)
def _agg2_call(hp_hbm, srcc_hbm, dstc_hbm, dinv_hbm, zer_hbm, s_hbm, g_hbm,
               sidx, didx, rows0, rows1, gs0, gs1,
               dtab, ghist, acc, shared, red, outb):
    cid = lax.axis_index("c")
    sid = lax.axis_index("s")
    wid = cid * NS + sid
    pltpu.sync_copy(zer_hbm, acc.at[pl.ds(sid * SL, SL)])
    pltpu.sync_copy(srcc_hbm.at[wid], sidx)
    pltpu.sync_copy(dstc_hbm.at[wid], didx)
    pltpu.sync_copy(dinv_hbm, dtab)
    _zero_vmem_1d(ghist, NPAD)
    plsc.subcore_barrier()

    def gpart(c):
        for j in range(CHE // 16):
            sl = pl.ds(j * 16, 16)
            vals = plsc.load_gather(dtab, [didx[c, sl]])
            plsc.addupdate_scatter(ghist, [sidx[c, sl]], vals)

    _ring_body(hp_hbm, sidx, didx, rows0, rows1, gs0, gs1, acc,
               gpart=gpart)
    plsc.subcore_barrier()
    pltpu.sync_copy(acc.at[pl.ds(sid * SL, SL)],
                    s_hbm.at[cid, pl.ds(sid * SL, SL)])
    _combine_and_store(ghist, shared, red, outb, g_hbm)


def _mm1_body(ca_ref, cb_ref, x_ref, w_ref, p_ref, dinv_ref):
    i = pl.program_id(0)
    row = lax.broadcasted_iota(jnp.int32, (BR, 1), 0) + i * BR
    deg = ca_ref[...] + cb_ref[...] + 1.0
    dv = jnp.where(row < N, lax.rsqrt(deg), 0.0)
    dinv_ref[...] = dv
    p_ref[...] = jnp.dot(x_ref[...], w_ref[...],
                         preferred_element_type=jnp.float32) * dv


def _mm2_body(sa_ref, sb_ref, p1_ref, dv_ref, b1_ref, w2_ref, out_ref):
    dv = dv_ref[...]
    h = dv * (sa_ref[...] + sb_ref[...] + p1_ref[...]) + b1_ref[...]
    h = jnp.maximum(h, 0.0)
    out_ref[...] = jnp.dot(h, w2_ref[...],
                           preferred_element_type=jnp.float32) * dv


def _fin_body(sa_ref, sb_ref, p2_ref, dv_ref, ga_ref, gb_ref, b2_ref, w3_ref,
              b3_ref, out_ref, acc_ref):
    i = pl.program_id(0)

    @pl.when(i == 0)
    def _z():
        acc_ref[...] = jnp.zeros_like(acc_ref)

    dv = dv_ref[...]
    h = jnp.maximum(dv * (sa_ref[...] + sb_ref[...] + p2_ref[...]) + b2_ref[...],
                    0.0)
    c = dv * (ga_ref[...] + gb_ref[...] + dv)
    acc_ref[...] += jnp.sum(c * h, axis=0, keepdims=True)

    @pl.when(i == GRID - 1)
    def _f():
        out_ref[...] = jnp.dot(acc_ref[...] * (1.0 / N), w3_ref[...],
                               preferred_element_type=jnp.float32) + b3_ref[...]


def _col_spec(d):
    return pl.BlockSpec((BR, d), lambda i: (i, 0))


def _const_spec(shape):
    return pl.BlockSpec(shape, lambda i: tuple(0 for _ in shape))


def kernel(x, edge_index, W1, b1, W2, b2, W3, b3):
    f32 = jnp.float32
    src = edge_index[0].astype(jnp.int32)
    dst = edge_index[1].astype(jnp.int32)
    npd = EPAD - E
    pidx = jnp.arange(npd, dtype=jnp.int32)
    srcp = jnp.concatenate([src, pidx % N])
    dstp = jnp.concatenate([dst, N + (pidx % (NPAD - N))])
    srcc = srcp.reshape(NT, NCHT, CHE)
    dstc = dstp.reshape(NT, NCHT, CHE)
    xpad = jnp.zeros((NPAD, D_IN), f32).at[:N].set(x)
    W1p = jnp.zeros((D_IN, D1P), f32).at[:, :D1].set(W1)
    b1p = jnp.zeros((1, D1P), f32).at[0, :D1].set(b1)
    W2p = jnp.zeros((D1P, D2), f32).at[:D1].set(W2)

    cnt2 = _deg_call(dstc)

    P1p, dinv = pl.pallas_call(
        _mm1_body,
        grid=(GRID,),
        in_specs=[_col_spec(1), _col_spec(1), _col_spec(D_IN),
                  _const_spec((D_IN, D1P))],
        out_specs=[_col_spec(D1P), _col_spec(1)],
        out_shape=[jax.ShapeDtypeStruct((NPAD, D1P), f32),
                   jax.ShapeDtypeStruct((NPAD, 1), f32)],
    )(cnt2[0][:, None], cnt2[1][:, None], xpad, W1p)

    z1 = jnp.zeros((SL, D1P), f32)
    S1 = _agg1_call(P1p, srcc, dstc, z1)

    P2p = pl.pallas_call(
        _mm2_body,
        grid=(GRID,),
        in_specs=[_col_spec(D1P), _col_spec(D1P), _col_spec(D1P), _col_spec(1),
                  _const_spec((1, D1P)), _const_spec((D1P, D2))],
        out_specs=_col_spec(D2),
        out_shape=jax.ShapeDtypeStruct((NPAD, D2), f32),
    )(S1[0], S1[1], P1p, dinv, b1p, W2p)

    z2 = jnp.zeros((SL, D2), f32)
    S2, g2 = _agg2_call(P2p, srcc, dstc, dinv[:, 0], z2)

    out = pl.pallas_call(
        _fin_body,
        grid=(GRID,),
        in_specs=[_col_spec(D2), _col_spec(D2), _col_spec(D2), _col_spec(1),
                  _col_spec(1), _col_spec(1), _const_spec((1, D2)),
                  _const_spec((D2, D3)), _const_spec((1, D3))],
        out_specs=_const_spec((1, D3)),
        out_shape=jax.ShapeDtypeStruct((1, D3), f32),
        scratch_shapes=[pltpu.VMEM((1, D2), f32)],
    )(S2[0], S2[1], P2p, dinv, g2[0][:, None], g2[1][:, None],
      b2[None, :], W3, b3[None, :])

    return out


# speedup vs baseline: 1.0364x; 1.0364x over previous
import functools

import jax
import jax.numpy as jnp
from jax import lax
from jax.experimental import pallas as pl
from jax.experimental.pallas import tpu as pltpu
from jax.experimental.pallas import tpu_sc as plsc

N = 10000
E = 160000
D_IN = 256
D1 = 100
D1P = 112
D2 = 64
D3 = 32

NC = 2
NS = 16
NT = NC * NS
NPAD = 10240
SL = NPAD // NS
CHE = 128
NCHT = 40
EPT = NCHT * CHE
EPAD = NT * EPT

BR = 1024
GRID = NPAD // BR

_mesh = plsc.VectorSubcoreMesh(
    core_axis_name="c", subcore_axis_name="s", num_cores=NC, num_subcores=NS)
_sc_params = pltpu.CompilerParams(
    needs_layout_passes=False, use_tc_tiling_on_sc=False)


def _wid():
    return lax.axis_index("c") * NS + lax.axis_index("s")


def _zero_vmem_1d(ref, n):
    z = jnp.zeros((16,), jnp.float32)

    def body(i, _):
        ref[pl.ds(i * 16, 16)] = z
        return _

    lax.fori_loop(0, n // 16, body, None)


def _combine_and_store(hist, shared, red, outb, out_hbm):
    sid = lax.axis_index("s")
    cid = lax.axis_index("c")
    pltpu.sync_copy(hist, shared.at[sid])
    plsc.subcore_barrier()
    for k in range(NS):
        pltpu.sync_copy(shared.at[k, pl.ds(sid * SL, SL)], red.at[k])

    def body(j, _):
        sl = pl.ds(j * 16, 16)
        acc = red[0, sl]
        for k in range(1, NS):
            acc = acc + red[k, sl]
        outb[sl] = acc
        return _

    lax.fori_loop(0, SL // 16, body, None)
    pltpu.sync_copy(outb, out_hbm.at[cid, pl.ds(sid * SL, SL)])


@functools.partial(
    pl.kernel,
    out_type=jax.ShapeDtypeStruct((NC, NPAD), jnp.float32),
    mesh=_mesh,
    compiler_params=_sc_params,
    scratch_types=[
        pltpu.VMEM((NPAD,), jnp.float32),
        pltpu.VMEM((NCHT, CHE), jnp.int32),
        pltpu.VMEM_SHARED((NS, NPAD), jnp.float32),
        pltpu.VMEM((NS, SL), jnp.float32),
        pltpu.VMEM((SL,), jnp.float32),
    ],
)
def _deg_call(dstc_hbm, out_hbm, hist, dif, shared, red, outb):
    wid = _wid()
    pltpu.sync_copy(dstc_hbm.at[wid], dif)
    _zero_vmem_1d(hist, NPAD)
    ones = jnp.ones((16,), jnp.float32)

    def body(i, _):
        def inner(j, _2):
            idx = dif[i, pl.ds(j * 16, 16)]
            plsc.addupdate_scatter(hist, [idx], ones)
            return _2

        lax.fori_loop(0, CHE // 16, inner, None)
        return _

    lax.fori_loop(0, NCHT, body, None)
    _combine_and_store(hist, shared, red, outb, out_hbm)


@functools.partial(
    pl.kernel,
    out_type=jax.ShapeDtypeStruct((NC, NPAD), jnp.float32),
    mesh=_mesh,
    compiler_params=_sc_params,
    scratch_types=[
        pltpu.VMEM((NPAD,), jnp.float32),
        pltpu.VMEM((NPAD,), jnp.float32),
        pltpu.VMEM((NCHT, CHE), jnp.int32),
        pltpu.VMEM((NCHT, CHE), jnp.int32),
        pltpu.VMEM_SHARED((NS, NPAD), jnp.float32),
        pltpu.VMEM((NS, SL), jnp.float32),
        pltpu.VMEM((SL,), jnp.float32),
    ],
)
def _g_call(dinv_hbm, srcc_hbm, dstc_hbm, out_hbm, dtab, hist, sif, dif,
            shared, red, outb):
    wid = _wid()
    pltpu.sync_copy(srcc_hbm.at[wid], sif)
    pltpu.sync_copy(dstc_hbm.at[wid], dif)
    pltpu.sync_copy(dinv_hbm, dtab)
    _zero_vmem_1d(hist, NPAD)

    def body(i, _):
        def inner(j, _2):
            sl = pl.ds(j * 16, 16)
            vals = plsc.load_gather(dtab, [dif[i, sl]])
            plsc.addupdate_scatter(hist, [sif[i, sl]], vals)
            return _2

        lax.fori_loop(0, CHE // 16, inner, None)
        return _

    lax.fori_loop(0, NCHT, body, None)
    _combine_and_store(hist, shared, red, outb, out_hbm)


def _ring_body(hp_hbm, sidx, didx, rows0, rows1, gs0, gs1, acc, gpart=None):
    pltpu.async_copy(hp_hbm.at[sidx.at[0]], rows0, gs0)

    def body(i, _):
        c0 = 2 * i
        c1 = c0 + 1
        pltpu.async_copy(hp_hbm.at[sidx.at[c1]], rows1, gs1)
        if gpart is not None:
            gpart(c0)
        pltpu.make_async_copy(hp_hbm.at[sidx.at[c0]], rows0, gs0).wait()
        pltpu.sync_copy(rows0, acc.at[didx.at[c0]], add=True)

        @pl.when(i < NCHT // 2 - 1)
        def _nx():
            pltpu.async_copy(hp_hbm.at[sidx.at[c0 + 2]], rows0, gs0)

        if gpart is not None:
            gpart(c1)
        pltpu.make_async_copy(hp_hbm.at[sidx.at[c1]], rows1, gs1).wait()
        pltpu.sync_copy(rows1, acc.at[didx.at[c1]], add=True)
        return _

    lax.fori_loop(0, NCHT // 2, body, None)


@functools.partial(
    pl.kernel,
    out_type=jax.ShapeDtypeStruct((NC, NPAD, D1P), jnp.float32),
    mesh=_mesh,
    compiler_params=_sc_params,
    scratch_types=[
        pltpu.VMEM((NCHT, CHE), jnp.int32),
        pltpu.VMEM((NCHT, CHE), jnp.int32),
        pltpu.VMEM((CHE, D1P), jnp.float32),
        pltpu.VMEM((CHE, D1P), jnp.float32),
        pltpu.SemaphoreType.DMA,
        pltpu.SemaphoreType.DMA,
        pltpu.VMEM_SHARED((NPAD, D1P), jnp.float32),
    ],
)
def _agg1_call(hp_hbm, srcc_hbm, dstc_hbm, zer_hbm, s_hbm, sidx, didx,
               rows0, rows1, gs0, gs1, acc):
    cid = lax.axis_index("c")
    sid = lax.axis_index("s")
    wid = cid * NS + sid
    pltpu.sync_copy(zer_hbm, acc.at[pl.ds(sid * SL, SL)])
    pltpu.sync_copy(srcc_hbm.at[wid], sidx)
    pltpu.sync_copy(dstc_hbm.at[wid], didx)
    plsc.subcore_barrier()
    _ring_body(hp_hbm, sidx, didx, rows0, rows1, gs0, gs1, acc)
    plsc.subcore_barrier()
    pltpu.sync_copy(acc.at[pl.ds(sid * SL, SL)],
                    s_hbm.at[cid, pl.ds(sid * SL, SL)])


@functools.partial(
    pl.kernel,
    out_type=jax.ShapeDtypeStruct((NC, NPAD, D2), jnp.float32),
    mesh=_mesh,
    compiler_params=_sc_params,
    scratch_types=[
        pltpu.VMEM((NCHT, CHE), jnp.int32),
        pltpu.VMEM((NCHT, CHE), jnp.int32),
        pltpu.VMEM((CHE, D2), jnp.float32),
        pltpu.VMEM((CHE, D2), jnp.float32),
        pltpu.SemaphoreType.DMA,
        pltpu.SemaphoreType.DMA,
        pltpu.VMEM_SHARED((NPAD, D2), jnp.float32),
    ],
)
def _agg2_call(hp_hbm, srcc_hbm, dstc_hbm, zer_hbm, s_hbm,
               sidx, didx, rows0, rows1, gs0, gs1, acc):
    cid = lax.axis_index("c")
    sid = lax.axis_index("s")
    wid = cid * NS + sid
    pltpu.sync_copy(zer_hbm, acc.at[pl.ds(sid * SL, SL)])
    pltpu.sync_copy(srcc_hbm.at[wid], sidx)
    pltpu.sync_copy(dstc_hbm.at[wid], didx)
    plsc.subcore_barrier()
    _ring_body(hp_hbm, sidx, didx, rows0, rows1, gs0, gs1, acc)
    plsc.subcore_barrier()
    pltpu.sync_copy(acc.at[pl.ds(sid * SL, SL)],
                    s_hbm.at[cid, pl.ds(sid * SL, SL)])


def _mm1_body(ca_ref, cb_ref, x_ref, w_ref, p_ref, dinv_ref):
    i = pl.program_id(0)
    row = lax.broadcasted_iota(jnp.int32, (BR, 1), 0) + i * BR
    deg = ca_ref[...] + cb_ref[...] + 1.0
    dv = jnp.where(row < N, lax.rsqrt(deg), 0.0)
    dinv_ref[...] = dv
    p_ref[...] = jnp.dot(x_ref[...], w_ref[...],
                         preferred_element_type=jnp.float32) * dv


def _mm2_body(sa_ref, sb_ref, p1_ref, dv_ref, b1_ref, w2_ref, out_ref):
    dv = dv_ref[...]
    h = dv * (sa_ref[...] + sb_ref[...] + p1_ref[...]) + b1_ref[...]
    h = jnp.maximum(h, 0.0)
    out_ref[...] = jnp.dot(h, w2_ref[...],
                           preferred_element_type=jnp.float32) * dv


def _fin_body(sa_ref, sb_ref, p2_ref, dv_ref, ga_ref, gb_ref, b2_ref, w3_ref,
              b3_ref, out_ref, acc_ref):
    i = pl.program_id(0)

    @pl.when(i == 0)
    def _z():
        acc_ref[...] = jnp.zeros_like(acc_ref)

    dv = dv_ref[...]
    h = jnp.maximum(dv * (sa_ref[...] + sb_ref[...] + p2_ref[...]) + b2_ref[...],
                    0.0)
    c = dv * (ga_ref[...] + gb_ref[...] + dv)
    acc_ref[...] += jnp.sum(c * h, axis=0, keepdims=True)

    @pl.when(i == GRID - 1)
    def _f():
        out_ref[...] = jnp.dot(acc_ref[...] * (1.0 / N), w3_ref[...],
                               preferred_element_type=jnp.float32) + b3_ref[...]


def _col_spec(d):
    return pl.BlockSpec((BR, d), lambda i: (i, 0))


def _const_spec(shape):
    return pl.BlockSpec(shape, lambda i: tuple(0 for _ in shape))


def kernel(x, edge_index, W1, b1, W2, b2, W3, b3):
    f32 = jnp.float32
    src = edge_index[0].astype(jnp.int32)
    dst = edge_index[1].astype(jnp.int32)
    npd = EPAD - E
    pidx = jnp.arange(npd, dtype=jnp.int32)
    srcp = jnp.concatenate([src, pidx % N])
    dstp = jnp.concatenate([dst, N + (pidx % (NPAD - N))])
    srcc = srcp.reshape(NT, NCHT, CHE)
    dstc = dstp.reshape(NT, NCHT, CHE)
    xpad = jnp.zeros((NPAD, D_IN), f32).at[:N].set(x)
    W1p = jnp.zeros((D_IN, D1P), f32).at[:, :D1].set(W1)
    b1p = jnp.zeros((1, D1P), f32).at[0, :D1].set(b1)
    W2p = jnp.zeros((D1P, D2), f32).at[:D1].set(W2)

    cnt2 = _deg_call(dstc)

    P1p, dinv = pl.pallas_call(
        _mm1_body,
        grid=(GRID,),
        in_specs=[_col_spec(1), _col_spec(1), _col_spec(D_IN),
                  _const_spec((D_IN, D1P))],
        out_specs=[_col_spec(D1P), _col_spec(1)],
        out_shape=[jax.ShapeDtypeStruct((NPAD, D1P), f32),
                   jax.ShapeDtypeStruct((NPAD, 1), f32)],
    )(cnt2[0][:, None], cnt2[1][:, None], xpad, W1p)

    z1 = jnp.zeros((SL, D1P), f32)
    g2 = _g_call(dinv[:, 0], srcc, dstc)
    S1 = _agg1_call(P1p, srcc, dstc, z1)

    P2p = pl.pallas_call(
        _mm2_body,
        grid=(GRID,),
        in_specs=[_col_spec(D1P), _col_spec(D1P), _col_spec(D1P), _col_spec(1),
                  _const_spec((1, D1P)), _const_spec((D1P, D2))],
        out_specs=_col_spec(D2),
        out_shape=jax.ShapeDtypeStruct((NPAD, D2), f32),
    )(S1[0], S1[1], P1p, dinv, b1p, W2p)

    z2 = jnp.zeros((SL, D2), f32)
    S2 = _agg2_call(P2p, srcc, dstc, z2)

    out = pl.pallas_call(
        _fin_body,
        grid=(GRID,),
        in_specs=[_col_spec(D2), _col_spec(D2), _col_spec(D2), _col_spec(1),
                  _col_spec(1), _col_spec(1), _const_spec((1, D2)),
                  _const_spec((D2, D3)), _const_spec((1, D3))],
        out_specs=_const_spec((1, D3)),
        out_shape=jax.ShapeDtypeStruct((1, D3), f32),
        scratch_shapes=[pltpu.VMEM((1, D2), f32)],
    )(S2[0], S2[1], P2p, dinv, g2[0][:, None], g2[1][:, None],
      b2[None, :], W3, b3[None, :])

    return out


# speedup vs baseline: 1.0683x; 1.0308x over previous
import functools

import jax
import jax.numpy as jnp
from jax import lax
from jax.experimental import pallas as pl
from jax.experimental.pallas import tpu as pltpu
from jax.experimental.pallas import tpu_sc as plsc

N = 10000
E = 160000
D_IN = 256
D1 = 100
D1P = 112
D2 = 64
D3 = 32

NC = 2
NS = 16
NT = NC * NS
NPAD = 10240
SL = NPAD // NS
CHE = 100
NCHC = 50
EPT = NCHC * CHE

BR = 1024
GRID = NPAD // BR

_mesh = plsc.VectorSubcoreMesh(
    core_axis_name="c", subcore_axis_name="s", num_cores=NC, num_subcores=NS)
_sc_params = pltpu.CompilerParams(
    needs_layout_passes=False, use_tc_tiling_on_sc=False)


def _wid():
    return lax.axis_index("c") * NS + lax.axis_index("s")


def _zero_vmem_1d(ref, n):
    z = jnp.zeros((16,), jnp.float32)

    def body(i, _):
        ref[pl.ds(i * 16, 16)] = z
        return _

    lax.fori_loop(0, n // 16, body, None)


def _combine_and_store(hist, shared, red, outb, out_hbm):
    sid = lax.axis_index("s")
    cid = lax.axis_index("c")
    pltpu.sync_copy(hist, shared.at[sid])
    plsc.subcore_barrier()
    for k in range(NS):
        pltpu.sync_copy(shared.at[k, pl.ds(sid * SL, SL)], red.at[k])

    def body(j, _):
        sl = pl.ds(j * 16, 16)
        acc = red[0, sl]
        for k in range(1, NS):
            acc = acc + red[k, sl]
        outb[sl] = acc
        return _

    lax.fori_loop(0, SL // 16, body, None)
    pltpu.sync_copy(outb, out_hbm.at[cid, pl.ds(sid * SL, SL)])


@functools.partial(
    pl.kernel,
    out_type=jax.ShapeDtypeStruct((NC, NPAD), jnp.float32),
    mesh=_mesh,
    compiler_params=_sc_params,
    scratch_types=[
        pltpu.VMEM((NPAD,), jnp.float32),
        pltpu.VMEM((EPT,), jnp.int32),
        pltpu.VMEM_SHARED((NS, NPAD), jnp.float32),
        pltpu.VMEM((NS, SL), jnp.float32),
        pltpu.VMEM((SL,), jnp.float32),
    ],
)
def _deg_call(eixf_hbm, out_hbm, hist, dif, shared, red, outb):
    wid = _wid()
    pltpu.sync_copy(eixf_hbm.at[1, wid], dif)
    _zero_vmem_1d(hist, NPAD)
    ones = jnp.ones((16,), jnp.float32)

    def body(i, _):
        idx = dif[pl.ds(i * 16, 16)]
        plsc.addupdate_scatter(hist, [idx], ones)
        return _

    lax.fori_loop(0, EPT // 16, body, None)
    if EPT % 16:
        tmask = lax.iota(jnp.int32, 16) >= (16 - EPT % 16)
        idx = dif[pl.ds(EPT - 16, 16)]
        plsc.addupdate_scatter(hist, [idx], ones, mask=tmask)
    _combine_and_store(hist, shared, red, outb, out_hbm)


@functools.partial(
    pl.kernel,
    out_type=jax.ShapeDtypeStruct((NC, NPAD), jnp.float32),
    mesh=_mesh,
    compiler_params=_sc_params,
    scratch_types=[
        pltpu.VMEM((NPAD,), jnp.float32),
        pltpu.VMEM((NPAD,), jnp.float32),
        pltpu.VMEM((EPT,), jnp.int32),
        pltpu.VMEM((EPT,), jnp.int32),
        pltpu.VMEM_SHARED((NS, NPAD), jnp.float32),
        pltpu.VMEM((NS, SL), jnp.float32),
        pltpu.VMEM((SL,), jnp.float32),
    ],
)
def _g_call(dinv_hbm, eixf_hbm, out_hbm, dtab, hist, sif, dif,
            shared, red, outb):
    wid = _wid()
    pltpu.sync_copy(eixf_hbm.at[0, wid], sif)
    pltpu.sync_copy(eixf_hbm.at[1, wid], dif)
    pltpu.sync_copy(dinv_hbm, dtab)
    _zero_vmem_1d(hist, NPAD)

    def body(i, _):
        sl = pl.ds(i * 16, 16)
        vals = plsc.load_gather(dtab, [dif[sl]])
        plsc.addupdate_scatter(hist, [sif[sl]], vals)
        return _

    lax.fori_loop(0, EPT // 16, body, None)
    if EPT % 16:
        tmask = lax.iota(jnp.int32, 16) >= (16 - EPT % 16)
        sl = pl.ds(EPT - 16, 16)
        vals = plsc.load_gather(dtab, [dif[sl]], mask=tmask)
        plsc.addupdate_scatter(hist, [sif[sl]], vals, mask=tmask)
    _combine_and_store(hist, shared, red, outb, out_hbm)


def _ring_body(hp_hbm, sidx, didx, rows0, rows1, gs0, gs1, acc):
    pltpu.async_copy(hp_hbm.at[sidx.at[0]], rows0, gs0)

    def body(i, _):
        c0 = 2 * i
        c1 = c0 + 1
        pltpu.async_copy(hp_hbm.at[sidx.at[c1]], rows1, gs1)
        pltpu.make_async_copy(hp_hbm.at[sidx.at[c0]], rows0, gs0).wait()
        pltpu.sync_copy(rows0, acc.at[didx.at[c0]], add=True)

        @pl.when(i < NCHC // 2 - 1)
        def _nx():
            pltpu.async_copy(hp_hbm.at[sidx.at[c0 + 2]], rows0, gs0)

        pltpu.make_async_copy(hp_hbm.at[sidx.at[c1]], rows1, gs1).wait()
        pltpu.sync_copy(rows1, acc.at[didx.at[c1]], add=True)
        return _

    lax.fori_loop(0, NCHC // 2, body, None)


def _make_agg(d):
    @functools.partial(
        pl.kernel,
        out_type=jax.ShapeDtypeStruct((NC, NPAD, d), jnp.float32),
        mesh=_mesh,
        compiler_params=_sc_params,
        scratch_types=[
            pltpu.VMEM((NCHC, CHE), jnp.int32),
            pltpu.VMEM((NCHC, CHE), jnp.int32),
            pltpu.VMEM((CHE, d), jnp.float32),
            pltpu.VMEM((CHE, d), jnp.float32),
            pltpu.SemaphoreType.DMA,
            pltpu.SemaphoreType.DMA,
            pltpu.VMEM_SHARED((NPAD, d), jnp.float32),
        ],
    )
    def agg(hp_hbm, eixc_hbm, zer_hbm, s_hbm, sidx, didx,
            rows0, rows1, gs0, gs1, acc):
        cid = lax.axis_index("c")
        sid = lax.axis_index("s")
        wid = cid * NS + sid
        pltpu.sync_copy(zer_hbm, acc.at[pl.ds(sid * SL, SL)])
        pltpu.sync_copy(eixc_hbm.at[0, wid], sidx)
        pltpu.sync_copy(eixc_hbm.at[1, wid], didx)
        plsc.subcore_barrier()
        _ring_body(hp_hbm, sidx, didx, rows0, rows1, gs0, gs1, acc)
        plsc.subcore_barrier()
        pltpu.sync_copy(acc.at[pl.ds(sid * SL, SL)],
                        s_hbm.at[cid, pl.ds(sid * SL, SL)])

    return agg


_agg1_call = _make_agg(D1P)
_agg2_call = _make_agg(D2)


def _mm1_body(cnt_ref, x_ref, w_ref, p_ref, dvc_ref, dv1_ref):
    i = pl.program_id(0)
    cb = cnt_ref[...]
    deg = cb[0:1, :] + cb[1:2, :] + 1.0
    col = lax.broadcasted_iota(jnp.int32, (1, BR), 1) + i * BR
    dvr = jnp.where(col < N, lax.rsqrt(deg), 0.0)
    dv1_ref[...] = dvr[0]
    dvc = jnp.transpose(dvr)
    dvc_ref[...] = dvc
    p_ref[...] = jnp.dot(x_ref[...], w_ref[...],
                         preferred_element_type=jnp.float32) * dvc


def _mm2_body(sa_ref, sb_ref, p1_ref, dv_ref, b1_ref, w2_ref, out_ref):
    dv = dv_ref[...]
    h = dv * (sa_ref[...] + sb_ref[...] + p1_ref[...]) + b1_ref[...]
    h = jnp.maximum(h, 0.0)
    out_ref[...] = jnp.dot(h, w2_ref[...],
                           preferred_element_type=jnp.float32) * dv


def _fin_body(sa_ref, sb_ref, p2_ref, dvc_ref, dv1_ref, g_ref, b2_ref,
              w3_ref, b3_ref, out_ref, acc_ref):
    i = pl.program_id(0)

    @pl.when(i == 0)
    def _z():
        acc_ref[...] = jnp.zeros_like(acc_ref)

    dvc = dvc_ref[...]
    h = jnp.maximum(
        dvc * (sa_ref[...] + sb_ref[...] + p2_ref[...]) + b2_ref[...], 0.0)
    gb = g_ref[...]
    dvr = dv1_ref[...][None, :]
    c = dvr * (gb[0:1, :] + gb[1:2, :] + dvr)
    acc_ref[...] += jnp.dot(c, h, preferred_element_type=jnp.float32)

    @pl.when(i == GRID - 1)
    def _f():
        out_ref[...] = jnp.dot(acc_ref[...] * (1.0 / N), w3_ref[...],
                               preferred_element_type=jnp.float32) + b3_ref[...]


def _col_spec(d):
    return pl.BlockSpec((BR, d), lambda i: (i, 0))


def _row_spec():
    return pl.BlockSpec((2, BR), lambda i: (0, i))


def _const_spec(shape):
    return pl.BlockSpec(shape, lambda i: tuple(0 for _ in shape))


def kernel(x, edge_index, W1, b1, W2, b2, W3, b3):
    f32 = jnp.float32
    ei32 = edge_index.astype(jnp.int32)
    eixc = ei32.reshape(2, NT, NCHC, CHE)
    eixf = ei32.reshape(2, NT, EPT)
    xpad = jnp.zeros((NPAD, D_IN), f32).at[:N].set(x)
    W1p = jnp.zeros((D_IN, D1P), f32).at[:, :D1].set(W1)
    b1p = jnp.zeros((1, D1P), f32).at[0, :D1].set(b1)
    W2p = jnp.zeros((D1P, D2), f32).at[:D1].set(W2)

    cnt2 = _deg_call(eixf)

    P1p, dinvc, dinv1 = pl.pallas_call(
        _mm1_body,
        grid=(GRID,),
        in_specs=[_row_spec(), _col_spec(D_IN), _const_spec((D_IN, D1P))],
        out_specs=[_col_spec(D1P), _col_spec(1),
                   pl.BlockSpec((BR,), lambda i: (i,))],
        out_shape=[jax.ShapeDtypeStruct((NPAD, D1P), f32),
                   jax.ShapeDtypeStruct((NPAD, 1), f32),
                   jax.ShapeDtypeStruct((NPAD,), f32)],
    )(cnt2, xpad, W1p)

    g2 = _g_call(dinv1, eixf)

    z1 = jnp.zeros((SL, D1P), f32)
    S1 = _agg1_call(P1p, eixc, z1)

    P2p = pl.pallas_call(
        _mm2_body,
        grid=(GRID,),
        in_specs=[_col_spec(D1P), _col_spec(D1P), _col_spec(D1P), _col_spec(1),
                  _const_spec((1, D1P)), _const_spec((D1P, D2))],
        out_specs=_col_spec(D2),
        out_shape=jax.ShapeDtypeStruct((NPAD, D2), f32),
    )(S1[0], S1[1], P1p, dinvc, b1p, W2p)

    z2 = jnp.zeros((SL, D2), f32)
    S2 = _agg2_call(P2p, eixc, z2)

    out = pl.pallas_call(
        _fin_body,
        grid=(GRID,),
        in_specs=[_col_spec(D2), _col_spec(D2), _col_spec(D2), _col_spec(1),
                  pl.BlockSpec((BR,), lambda i: (i,)), _row_spec(),
                  _const_spec((1, D2)), _const_spec((D2, D3)),
                  _const_spec((1, D3))],
        out_specs=_const_spec((1, D3)),
        out_shape=jax.ShapeDtypeStruct((1, D3), f32),
        scratch_shapes=[pltpu.VMEM((1, D2), f32)],
    )(S2[0], S2[1], P2p, dinvc, dinv1, g2, b2[None, :], W3, b3[None, :])

    return out


# speedup vs baseline: 1.1600x; 1.0858x over previous
import functools

import jax
import jax.numpy as jnp
from jax import lax
from jax.experimental import pallas as pl
from jax.experimental.pallas import tpu as pltpu
from jax.experimental.pallas import tpu_sc as plsc

N = 10000
E = 160000
D_IN = 256
D1 = 100
D1P = 112
D2 = 64
D3 = 32

NC = 2
NS = 16
NT = NC * NS
NPAD = 10240
SL = NPAD // NS
CHE = 100
NCHC = 50
EPT = NCHC * CHE

BR = 1024
GRID = NPAD // BR

_mesh = plsc.VectorSubcoreMesh(
    core_axis_name="c", subcore_axis_name="s", num_cores=NC, num_subcores=NS)
_sc_params = pltpu.CompilerParams(
    needs_layout_passes=False, use_tc_tiling_on_sc=False)


def _wid():
    return lax.axis_index("c") * NS + lax.axis_index("s")


def _zero_vmem_1d(ref, n):
    z = jnp.zeros((16,), jnp.float32)

    def body(i, _):
        ref[pl.ds(i * 16, 16)] = z
        return _

    lax.fori_loop(0, n // 16, body, None)


def _combine_and_store(hist, shared, red, outb, out_hbm):
    sid = lax.axis_index("s")
    cid = lax.axis_index("c")
    pltpu.sync_copy(hist, shared.at[sid])
    plsc.subcore_barrier()
    for k in range(NS):
        pltpu.sync_copy(shared.at[k, pl.ds(sid * SL, SL)], red.at[k])

    def body(j, _):
        sl = pl.ds(j * 16, 16)
        acc = red[0, sl]
        for k in range(1, NS):
            acc = acc + red[k, sl]
        outb[sl] = acc
        return _

    lax.fori_loop(0, SL // 16, body, None)
    pltpu.sync_copy(outb, out_hbm.at[cid, pl.ds(sid * SL, SL)])


@functools.partial(
    pl.kernel,
    out_type=jax.ShapeDtypeStruct((NC, NPAD), jnp.float32),
    mesh=_mesh,
    compiler_params=_sc_params,
    scratch_types=[
        pltpu.VMEM((NPAD,), jnp.float32),
        pltpu.VMEM((NCHC, CHE), jnp.int32),
        pltpu.VMEM_SHARED((NS, NPAD), jnp.float32),
        pltpu.VMEM((NS, SL), jnp.float32),
        pltpu.VMEM((SL,), jnp.float32),
    ],
)
def _deg_call(eixc_hbm, out_hbm, hist, dif, shared, red, outb):
    wid = _wid()
    pltpu.sync_copy(eixc_hbm.at[1, wid], dif)
    _zero_vmem_1d(hist, NPAD)
    ones = jnp.ones((16,), jnp.float32)
    tmask = lax.iota(jnp.int32, 16) >= (16 - CHE % 16)

    def body(c, _):
        for j in range(CHE // 16):
            idx = dif[c, pl.ds(j * 16, 16)]
            plsc.addupdate_scatter(hist, [idx], ones)
        idx = dif[c, pl.ds(CHE - 16, 16)]
        plsc.addupdate_scatter(hist, [idx], ones, mask=tmask)
        return _

    lax.fori_loop(0, NCHC, body, None)
    _combine_and_store(hist, shared, red, outb, out_hbm)


@functools.partial(
    pl.kernel,
    out_type=jax.ShapeDtypeStruct((NC, NPAD), jnp.float32),
    mesh=_mesh,
    compiler_params=_sc_params,
    scratch_types=[
        pltpu.VMEM((NPAD,), jnp.float32),
        pltpu.VMEM((NPAD,), jnp.float32),
        pltpu.VMEM((NCHC, CHE), jnp.int32),
        pltpu.VMEM((NCHC, CHE), jnp.int32),
        pltpu.VMEM_SHARED((NS, NPAD), jnp.float32),
        pltpu.VMEM((NS, SL), jnp.float32),
        pltpu.VMEM((SL,), jnp.float32),
    ],
)
def _g_call(dinv_hbm, eixc_hbm, out_hbm, dtab, hist, sif, dif,
            shared, red, outb):
    wid = _wid()
    pltpu.sync_copy(eixc_hbm.at[0, wid], sif)
    pltpu.sync_copy(eixc_hbm.at[1, wid], dif)
    pltpu.sync_copy(dinv_hbm, dtab)
    _zero_vmem_1d(hist, NPAD)
    tmask = lax.iota(jnp.int32, 16) >= (16 - CHE % 16)

    def body(c, _):
        for j in range(CHE // 16):
            sl = pl.ds(j * 16, 16)
            vals = plsc.load_gather(dtab, [dif[c, sl]])
            plsc.addupdate_scatter(hist, [sif[c, sl]], vals)
        sl = pl.ds(CHE - 16, 16)
        vals = plsc.load_gather(dtab, [dif[c, sl]], mask=tmask)
        plsc.addupdate_scatter(hist, [sif[c, sl]], vals, mask=tmask)
        return _

    lax.fori_loop(0, NCHC, body, None)
    _combine_and_store(hist, shared, red, outb, out_hbm)


def _ring_body(hp_hbm, sidx, didx, rows0, rows1, gs0, gs1, acc):
    pltpu.async_copy(hp_hbm.at[sidx.at[0]], rows0, gs0)

    def body(i, _):
        c0 = 2 * i
        c1 = c0 + 1
        pltpu.async_copy(hp_hbm.at[sidx.at[c1]], rows1, gs1)
        pltpu.make_async_copy(hp_hbm.at[sidx.at[c0]], rows0, gs0).wait()
        pltpu.sync_copy(rows0, acc.at[didx.at[c0]], add=True)

        @pl.when(i < NCHC // 2 - 1)
        def _nx():
            pltpu.async_copy(hp_hbm.at[sidx.at[c0 + 2]], rows0, gs0)

        pltpu.make_async_copy(hp_hbm.at[sidx.at[c1]], rows1, gs1).wait()
        pltpu.sync_copy(rows1, acc.at[didx.at[c1]], add=True)
        return _

    lax.fori_loop(0, NCHC // 2, body, None)


def _make_agg(d):
    @functools.partial(
        pl.kernel,
        out_type=jax.ShapeDtypeStruct((NC, NPAD, d), jnp.float32),
        mesh=_mesh,
        compiler_params=_sc_params,
        scratch_types=[
            pltpu.VMEM((NCHC, CHE), jnp.int32),
            pltpu.VMEM((NCHC, CHE), jnp.int32),
            pltpu.VMEM((CHE, d), jnp.float32),
            pltpu.VMEM((CHE, d), jnp.float32),
            pltpu.SemaphoreType.DMA,
            pltpu.SemaphoreType.DMA,
            pltpu.VMEM_SHARED((NPAD, d), jnp.float32),
        ],
    )
    def agg(hp_hbm, eixc_hbm, zer_hbm, s_hbm, sidx, didx,
            rows0, rows1, gs0, gs1, acc):
        cid = lax.axis_index("c")
        sid = lax.axis_index("s")
        wid = cid * NS + sid
        pltpu.sync_copy(zer_hbm, acc.at[pl.ds(sid * SL, SL)])
        pltpu.sync_copy(eixc_hbm.at[0, wid], sidx)
        pltpu.sync_copy(eixc_hbm.at[1, wid], didx)
        plsc.subcore_barrier()
        _ring_body(hp_hbm, sidx, didx, rows0, rows1, gs0, gs1, acc)
        plsc.subcore_barrier()
        pltpu.sync_copy(acc.at[pl.ds(sid * SL, SL)],
                        s_hbm.at[cid, pl.ds(sid * SL, SL)])

    return agg


_agg1_call = _make_agg(D1P)
_agg2_call = _make_agg(D2)


def _mm1_body(cnt_ref, x_ref, w_ref, p_ref, dvc_ref, dv1_ref):
    i = pl.program_id(0)
    cb = cnt_ref[...]
    deg = cb[0:1, :] + cb[1:2, :] + 1.0
    col = lax.broadcasted_iota(jnp.int32, (1, BR), 1) + i * BR
    dvr = jnp.where(col < N, lax.rsqrt(deg), 0.0)
    dv1_ref[...] = dvr[0]
    dvc = jnp.transpose(dvr)
    dvc_ref[...] = dvc
    p_ref[...] = jnp.dot(x_ref[...], w_ref[...],
                         preferred_element_type=jnp.float32) * dvc


def _mm2_body(s_ref, p1_ref, dv_ref, b1_ref, w2_ref, out_ref):
    dv = dv_ref[...]
    h = dv * (s_ref[0] + s_ref[1] + p1_ref[...]) + b1_ref[...]
    h = jnp.maximum(h, 0.0)
    out_ref[...] = jnp.dot(h, w2_ref[...],
                           preferred_element_type=jnp.float32) * dv


def _fin_body(s_ref, p2_ref, dvc_ref, dv1_ref, g_ref, b2_ref,
              w3_ref, b3_ref, out_ref, acc_ref):
    i = pl.program_id(0)

    @pl.when(i == 0)
    def _z():
        acc_ref[...] = jnp.zeros_like(acc_ref)

    dvc = dvc_ref[...]
    h = jnp.maximum(
        dvc * (s_ref[0] + s_ref[1] + p2_ref[...]) + b2_ref[...], 0.0)
    gb = g_ref[...]
    dvr = dv1_ref[...][None, :]
    c = dvr * (gb[0:1, :] + gb[1:2, :] + dvr)
    acc_ref[...] += jnp.dot(c, h, preferred_element_type=jnp.float32)

    @pl.when(i == GRID - 1)
    def _f():
        out_ref[...] = jnp.dot(acc_ref[...] * (1.0 / N), w3_ref[...],
                               preferred_element_type=jnp.float32) + b3_ref[...]


def _col_spec(d):
    return pl.BlockSpec((BR, d), lambda i: (i, 0))


def _row_spec():
    return pl.BlockSpec((2, BR), lambda i: (0, i))


def _s_spec(d):
    return pl.BlockSpec((2, BR, d), lambda i: (0, i, 0))


def _const_spec(shape):
    return pl.BlockSpec(shape, lambda i: tuple(0 for _ in shape))


def kernel(x, edge_index, W1, b1, W2, b2, W3, b3):
    f32 = jnp.float32
    ei32 = edge_index.astype(jnp.int32)
    eixc = ei32.reshape(2, NT, NCHC, CHE)
    xpad = jnp.zeros((NPAD, D_IN), f32).at[:N].set(x)
    W1p = jnp.zeros((D_IN, D1P), f32).at[:, :D1].set(W1)
    b1p = jnp.zeros((1, D1P), f32).at[0, :D1].set(b1)
    W2p = jnp.zeros((D1P, D2), f32).at[:D1].set(W2)

    cnt2 = _deg_call(eixc)

    P1p, dinvc, dinv1 = pl.pallas_call(
        _mm1_body,
        grid=(GRID,),
        in_specs=[_row_spec(), _col_spec(D_IN), _const_spec((D_IN, D1P))],
        out_specs=[_col_spec(D1P), _col_spec(1),
                   pl.BlockSpec((BR,), lambda i: (i,))],
        out_shape=[jax.ShapeDtypeStruct((NPAD, D1P), f32),
                   jax.ShapeDtypeStruct((NPAD, 1), f32),
                   jax.ShapeDtypeStruct((NPAD,), f32)],
    )(cnt2, xpad, W1p)

    g2 = _g_call(dinv1, eixc)

    z1 = jnp.zeros((SL, D1P), f32)
    S1 = _agg1_call(P1p, eixc, z1)

    P2p = pl.pallas_call(
        _mm2_body,
        grid=(GRID,),
        in_specs=[_s_spec(D1P), _col_spec(D1P), _col_spec(1),
                  _const_spec((1, D1P)), _const_spec((D1P, D2))],
        out_specs=_col_spec(D2),
        out_shape=jax.ShapeDtypeStruct((NPAD, D2), f32),
    )(S1, P1p, dinvc, b1p, W2p)

    z2 = jnp.zeros((SL, D2), f32)
    S2 = _agg2_call(P2p, eixc, z2)

    out = pl.pallas_call(
        _fin_body,
        grid=(GRID,),
        in_specs=[_s_spec(D2), _col_spec(D2), _col_spec(1),
                  pl.BlockSpec((BR,), lambda i: (i,)), _row_spec(),
                  _const_spec((1, D2)), _const_spec((D2, D3)),
                  _const_spec((1, D3))],
        out_specs=_const_spec((1, D3)),
        out_shape=jax.ShapeDtypeStruct((1, D3), f32),
        scratch_shapes=[pltpu.VMEM((1, D2), f32)],
    )(S2, P2p, dinvc, dinv1, g2, b2[None, :], W3, b3[None, :])

    return out


# speedup vs baseline: 1.2055x; 1.0392x over previous
import functools

import jax
import jax.numpy as jnp
from jax import lax
from jax.experimental import pallas as pl
from jax.experimental.pallas import tpu as pltpu
from jax.experimental.pallas import tpu_sc as plsc

N = 10000
E = 160000
D_IN = 256
D1 = 100
D1P = 112
D2 = 64
D3 = 32

NC = 2
NS = 16
NT = NC * NS
NPAD = 10240
SL = NPAD // NS
CHE = 100
NCHC = 50
EPT = NCHC * CHE

BR = 1024
GRID = NPAD // BR

_mesh = plsc.VectorSubcoreMesh(
    core_axis_name="c", subcore_axis_name="s", num_cores=NC, num_subcores=NS)
_sc_params = pltpu.CompilerParams(
    needs_layout_passes=False, use_tc_tiling_on_sc=False)


def _wid():
    return lax.axis_index("c") * NS + lax.axis_index("s")


def _zero_vmem_1d(ref, n):
    z = jnp.zeros((16,), jnp.float32)

    def body(i, _):
        ref[pl.ds(i * 16, 16)] = z
        return _

    lax.fori_loop(0, n // 16, body, None)


def _combine_and_store(hist, shared, red, outb, out_hbm):
    sid = lax.axis_index("s")
    cid = lax.axis_index("c")
    pltpu.sync_copy(hist, shared.at[sid])
    plsc.subcore_barrier()
    for k in range(NS):
        pltpu.sync_copy(shared.at[k, pl.ds(sid * SL, SL)], red.at[k])

    def body(j, _):
        sl = pl.ds(j * 16, 16)
        acc = red[0, sl]
        for k in range(1, NS):
            acc = acc + red[k, sl]
        outb[sl] = acc
        return _

    lax.fori_loop(0, SL // 16, body, None)
    pltpu.sync_copy(outb, out_hbm.at[cid, pl.ds(sid * SL, SL)])


@functools.partial(
    pl.kernel,
    out_type=jax.ShapeDtypeStruct((NC, NPAD), jnp.float32),
    mesh=_mesh,
    compiler_params=_sc_params,
    scratch_types=[
        pltpu.VMEM((NPAD,), jnp.float32),
        pltpu.VMEM((NCHC, CHE), jnp.int32),
        pltpu.VMEM_SHARED((NS, NPAD), jnp.float32),
        pltpu.VMEM((NS, SL), jnp.float32),
        pltpu.VMEM((SL,), jnp.float32),
    ],
)
def _deg_call(eixc_hbm, out_hbm, hist, dif, shared, red, outb):
    wid = _wid()
    pltpu.sync_copy(eixc_hbm.at[1, wid], dif)
    _zero_vmem_1d(hist, NPAD)
    ones = jnp.ones((16,), jnp.float32)
    tmask = lax.iota(jnp.int32, 16) >= (16 - CHE % 16)

    def body(c, _):
        for j in range(CHE // 16):
            idx = dif[c, pl.ds(j * 16, 16)]
            plsc.addupdate_scatter(hist, [idx], ones)
        idx = dif[c, pl.ds(CHE - 16, 16)]
        plsc.addupdate_scatter(hist, [idx], ones, mask=tmask)
        return _

    lax.fori_loop(0, NCHC, body, None)
    _combine_and_store(hist, shared, red, outb, out_hbm)


@functools.partial(
    pl.kernel,
    out_type=jax.ShapeDtypeStruct((NC, NPAD), jnp.float32),
    mesh=_mesh,
    compiler_params=_sc_params,
    scratch_types=[
        pltpu.VMEM((NPAD,), jnp.float32),
        pltpu.VMEM((NPAD,), jnp.float32),
        pltpu.VMEM((NCHC, CHE), jnp.int32),
        pltpu.VMEM((NCHC, CHE), jnp.int32),
        pltpu.VMEM_SHARED((NS, NPAD), jnp.float32),
        pltpu.VMEM((NS, SL), jnp.float32),
        pltpu.VMEM((SL,), jnp.float32),
    ],
)
def _g_call(dinv_hbm, eixc_hbm, out_hbm, dtab, hist, sif, dif,
            shared, red, outb):
    wid = _wid()
    pltpu.sync_copy(eixc_hbm.at[0, wid], sif)
    pltpu.sync_copy(eixc_hbm.at[1, wid], dif)
    pltpu.sync_copy(dinv_hbm, dtab)
    _zero_vmem_1d(hist, NPAD)
    tmask = lax.iota(jnp.int32, 16) >= (16 - CHE % 16)

    def body(c, _):
        for j in range(CHE // 16):
            sl = pl.ds(j * 16, 16)
            vals = plsc.load_gather(dtab, [dif[c, sl]])
            plsc.addupdate_scatter(hist, [sif[c, sl]], vals)
        sl = pl.ds(CHE - 16, 16)
        vals = plsc.load_gather(dtab, [dif[c, sl]], mask=tmask)
        plsc.addupdate_scatter(hist, [sif[c, sl]], vals, mask=tmask)
        return _

    lax.fori_loop(0, NCHC, body, None)
    _combine_and_store(hist, shared, red, outb, out_hbm)


def _ring_body(hp_hbm, sidx, didx, rows0, rows1, gs0, gs1, acc):
    pltpu.async_copy(hp_hbm.at[sidx.at[0]], rows0, gs0)

    def body(i, _):
        c0 = 2 * i
        c1 = c0 + 1
        pltpu.async_copy(hp_hbm.at[sidx.at[c1]], rows1, gs1)
        pltpu.make_async_copy(hp_hbm.at[sidx.at[c0]], rows0, gs0).wait()
        pltpu.sync_copy(rows0, acc.at[didx.at[c0]], add=True)

        @pl.when(i < NCHC // 2 - 1)
        def _nx():
            pltpu.async_copy(hp_hbm.at[sidx.at[c0 + 2]], rows0, gs0)

        pltpu.make_async_copy(hp_hbm.at[sidx.at[c1]], rows1, gs1).wait()
        pltpu.sync_copy(rows1, acc.at[didx.at[c1]], add=True)
        return _

    lax.fori_loop(0, NCHC // 2, body, None)


def _make_agg(d):
    @functools.partial(
        pl.kernel,
        out_type=jax.ShapeDtypeStruct((NC, NPAD, d), jnp.float32),
        mesh=_mesh,
        compiler_params=_sc_params,
        scratch_types=[
            pltpu.VMEM((NCHC, CHE), jnp.int32),
            pltpu.VMEM((NCHC, CHE), jnp.int32),
            pltpu.VMEM((CHE, d), jnp.float32),
            pltpu.VMEM((CHE, d), jnp.float32),
            pltpu.SemaphoreType.DMA,
            pltpu.SemaphoreType.DMA,
            pltpu.VMEM_SHARED((NPAD, d), jnp.float32),
        ],
    )
    def agg(hp_hbm, eixc_hbm, s_hbm, sidx, didx,
            rows0, rows1, gs0, gs1, acc):
        cid = lax.axis_index("c")
        sid = lax.axis_index("s")
        wid = cid * NS + sid
        pltpu.sync_copy(eixc_hbm.at[0, wid], sidx)
        pltpu.sync_copy(eixc_hbm.at[1, wid], didx)
        z = jnp.zeros((16,), jnp.float32)

        def zbody(r, _):
            for j in range(d // 16):
                rows0[r, pl.ds(j * 16, 16)] = z
            return _

        lax.fori_loop(0, 80, zbody, None)
        for q in range(SL // 80):
            pltpu.sync_copy(rows0.at[pl.ds(0, 80)],
                            acc.at[pl.ds(sid * SL + q * 80, 80)])
        plsc.subcore_barrier()
        _ring_body(hp_hbm, sidx, didx, rows0, rows1, gs0, gs1, acc)
        plsc.subcore_barrier()
        pltpu.sync_copy(acc.at[pl.ds(sid * SL, SL)],
                        s_hbm.at[cid, pl.ds(sid * SL, SL)])

    return agg


_agg1_call = _make_agg(D1P)
_agg2_call = _make_agg(D2)


def _mm1_body(cnt_ref, x_ref, w_ref, p_ref, dvc_ref, dv1_ref):
    i = pl.program_id(0)
    cb = cnt_ref[...]
    deg = cb[0:1, :] + cb[1:2, :] + 1.0
    col = lax.broadcasted_iota(jnp.int32, (1, BR), 1) + i * BR
    dvr = jnp.where(col < N, lax.rsqrt(deg), 0.0)
    dv1_ref[...] = dvr[0]
    dvc = jnp.transpose(dvr)
    dvc_ref[...] = dvc
    p_ref[...] = jnp.dot(x_ref[...], w_ref[...],
                         preferred_element_type=jnp.float32) * dvc


def _mm2_body(s_ref, p1_ref, dv_ref, b1_ref, w2_ref, out_ref):
    dv = dv_ref[...]
    h = dv * (s_ref[0] + s_ref[1] + p1_ref[...]) + b1_ref[...]
    h = jnp.maximum(h, 0.0)
    out_ref[...] = jnp.dot(h, w2_ref[...],
                           preferred_element_type=jnp.float32) * dv


def _fin_body(s_ref, p2_ref, dvc_ref, dv1_ref, g_ref, b2_ref,
              w3_ref, b3_ref, out_ref, acc_ref):
    i = pl.program_id(0)

    @pl.when(i == 0)
    def _z():
        acc_ref[...] = jnp.zeros_like(acc_ref)

    dvc = dvc_ref[...]
    h = jnp.maximum(
        dvc * (s_ref[0] + s_ref[1] + p2_ref[...]) + b2_ref[...], 0.0)
    gb = g_ref[...]
    dvr = dv1_ref[...][None, :]
    c = dvr * (gb[0:1, :] + gb[1:2, :] + dvr)
    acc_ref[...] += jnp.dot(c, h, preferred_element_type=jnp.float32)

    @pl.when(i == GRID - 1)
    def _f():
        out_ref[...] = jnp.dot(acc_ref[...] * (1.0 / N), w3_ref[...],
                               preferred_element_type=jnp.float32) + b3_ref[...]


def _col_spec(d):
    return pl.BlockSpec((BR, d), lambda i: (i, 0))


def _row_spec():
    return pl.BlockSpec((2, BR), lambda i: (0, i))


def _s_spec(d):
    return pl.BlockSpec((2, BR, d), lambda i: (0, i, 0))


def _const_spec(shape):
    return pl.BlockSpec(shape, lambda i: tuple(0 for _ in shape))


def kernel(x, edge_index, W1, b1, W2, b2, W3, b3):
    f32 = jnp.float32
    ei32 = edge_index.astype(jnp.int32)
    eixc = ei32.reshape(2, NT, NCHC, CHE)
    xpad = jnp.zeros((NPAD, D_IN), f32).at[:N].set(x)
    W1p = jnp.zeros((D_IN, D1P), f32).at[:, :D1].set(W1)
    b1p = jnp.zeros((1, D1P), f32).at[0, :D1].set(b1)
    W2p = jnp.zeros((D1P, D2), f32).at[:D1].set(W2)

    cnt2 = _deg_call(eixc)

    P1p, dinvc, dinv1 = pl.pallas_call(
        _mm1_body,
        grid=(GRID,),
        in_specs=[_row_spec(), _col_spec(D_IN), _const_spec((D_IN, D1P))],
        out_specs=[_col_spec(D1P), _col_spec(1),
                   pl.BlockSpec((BR,), lambda i: (i,))],
        out_shape=[jax.ShapeDtypeStruct((NPAD, D1P), f32),
                   jax.ShapeDtypeStruct((NPAD, 1), f32),
                   jax.ShapeDtypeStruct((NPAD,), f32)],
    )(cnt2, xpad, W1p)

    g2 = _g_call(dinv1, eixc)

    S1 = _agg1_call(P1p, eixc)

    P2p = pl.pallas_call(
        _mm2_body,
        grid=(GRID,),
        in_specs=[_s_spec(D1P), _col_spec(D1P), _col_spec(1),
                  _const_spec((1, D1P)), _const_spec((D1P, D2))],
        out_specs=_col_spec(D2),
        out_shape=jax.ShapeDtypeStruct((NPAD, D2), f32),
    )(S1, P1p, dinvc, b1p, W2p)

    S2 = _agg2_call(P2p, eixc)

    out = pl.pallas_call(
        _fin_body,
        grid=(GRID,),
        in_specs=[_s_spec(D2), _col_spec(D2), _col_spec(1),
                  pl.BlockSpec((BR,), lambda i: (i,)), _row_spec(),
                  _const_spec((1, D2)), _const_spec((D2, D3)),
                  _const_spec((1, D3))],
        out_specs=_const_spec((1, D3)),
        out_shape=jax.ShapeDtypeStruct((1, D3), f32),
        scratch_shapes=[pltpu.VMEM((1, D2), f32)],
    )(S2, P2p, dinvc, dinv1, g2, b2[None, :], W3, b3[None, :])

    return out


# speedup vs baseline: 1.2079x; 1.0020x over previous
import functools

import jax
import jax.numpy as jnp
from jax import lax
from jax.experimental import pallas as pl
from jax.experimental.pallas import tpu as pltpu
from jax.experimental.pallas import tpu_sc as plsc

N = 10000
E = 160000
D_IN = 256
D1 = 100
D1P = 112
D2 = 64
D3 = 32

NC = 2
NS = 16
NT = NC * NS
NPAD = 10240
SL = NPAD // NS
CHE = 100
NCHC = 50
EPT = NCHC * CHE

BR = 1024
GRID = NPAD // BR

_mesh = plsc.VectorSubcoreMesh(
    core_axis_name="c", subcore_axis_name="s", num_cores=NC, num_subcores=NS)
_sc_params = pltpu.CompilerParams(
    needs_layout_passes=False, use_tc_tiling_on_sc=False)


def _wid():
    return lax.axis_index("c") * NS + lax.axis_index("s")


def _zero_vmem_1d(ref, n):
    z = jnp.zeros((16,), jnp.float32)

    def body(i, _):
        ref[pl.ds(i * 16, 16)] = z
        return _

    lax.fori_loop(0, n // 16, body, None)


def _combine_and_store(hist, shared, red, outb, out_hbm):
    sid = lax.axis_index("s")
    cid = lax.axis_index("c")
    pltpu.sync_copy(hist, shared.at[sid])
    plsc.subcore_barrier()
    pltpu.sync_copy(shared.at[:, pl.ds(sid * SL, SL)], red)

    def body(j, _):
        sl = pl.ds(j * 16, 16)
        acc = red[0, sl]
        for k in range(1, NS):
            acc = acc + red[k, sl]
        outb[sl] = acc
        return _

    lax.fori_loop(0, SL // 16, body, None)
    pltpu.sync_copy(outb, out_hbm.at[cid, pl.ds(sid * SL, SL)])


@functools.partial(
    pl.kernel,
    out_type=jax.ShapeDtypeStruct((NC, NPAD), jnp.float32),
    mesh=_mesh,
    compiler_params=_sc_params,
    scratch_types=[
        pltpu.VMEM((NPAD,), jnp.float32),
        pltpu.VMEM((NCHC, CHE), jnp.int32),
        pltpu.VMEM_SHARED((NS, NPAD), jnp.float32),
        pltpu.VMEM((NS, SL), jnp.float32),
        pltpu.VMEM((SL,), jnp.float32),
    ],
)
def _deg_call(eixc_hbm, out_hbm, hist, dif, shared, red, outb):
    wid = _wid()
    pltpu.sync_copy(eixc_hbm.at[1, wid], dif)
    _zero_vmem_1d(hist, NPAD)
    ones = jnp.ones((16,), jnp.float32)
    tmask = lax.iota(jnp.int32, 16) >= (16 - CHE % 16)

    def body(c, _):
        for j in range(CHE // 16):
            idx = dif[c, pl.ds(j * 16, 16)]
            plsc.addupdate_scatter(hist, [idx], ones)
        idx = dif[c, pl.ds(CHE - 16, 16)]
        plsc.addupdate_scatter(hist, [idx], ones, mask=tmask)
        return _

    lax.fori_loop(0, NCHC, body, None)
    _combine_and_store(hist, shared, red, outb, out_hbm)


@functools.partial(
    pl.kernel,
    out_type=jax.ShapeDtypeStruct((NC, NPAD), jnp.float32),
    mesh=_mesh,
    compiler_params=_sc_params,
    scratch_types=[
        pltpu.VMEM((NPAD,), jnp.float32),
        pltpu.VMEM((NPAD,), jnp.float32),
        pltpu.VMEM((NCHC, CHE), jnp.int32),
        pltpu.VMEM((NCHC, CHE), jnp.int32),
        pltpu.VMEM_SHARED((NS, NPAD), jnp.float32),
        pltpu.VMEM((NS, SL), jnp.float32),
        pltpu.VMEM((SL,), jnp.float32),
    ],
)
def _g_call(dinv_hbm, eixc_hbm, out_hbm, dtab, hist, sif, dif,
            shared, red, outb):
    wid = _wid()
    pltpu.sync_copy(eixc_hbm.at[0, wid], sif)
    pltpu.sync_copy(eixc_hbm.at[1, wid], dif)
    pltpu.sync_copy(dinv_hbm, dtab)
    _zero_vmem_1d(hist, NPAD)
    tmask = lax.iota(jnp.int32, 16) >= (16 - CHE % 16)

    def body(c, _):
        for j in range(CHE // 16):
            sl = pl.ds(j * 16, 16)
            vals = plsc.load_gather(dtab, [dif[c, sl]])
            plsc.addupdate_scatter(hist, [sif[c, sl]], vals)
        sl = pl.ds(CHE - 16, 16)
        vals = plsc.load_gather(dtab, [dif[c, sl]], mask=tmask)
        plsc.addupdate_scatter(hist, [sif[c, sl]], vals, mask=tmask)
        return _

    lax.fori_loop(0, NCHC, body, None)
    _combine_and_store(hist, shared, red, outb, out_hbm)


def _ring_body(hp_hbm, sidx, didx, rows0, rows1, gs0, gs1, acc):
    pltpu.async_copy(hp_hbm.at[sidx.at[0]], rows0, gs0)

    def body(i, _):
        c0 = 2 * i
        c1 = c0 + 1
        pltpu.async_copy(hp_hbm.at[sidx.at[c1]], rows1, gs1)
        pltpu.make_async_copy(hp_hbm.at[sidx.at[c0]], rows0, gs0).wait()
        pltpu.sync_copy(rows0, acc.at[didx.at[c0]], add=True)

        @pl.when(i < NCHC // 2 - 1)
        def _nx():
            pltpu.async_copy(hp_hbm.at[sidx.at[c0 + 2]], rows0, gs0)

        pltpu.make_async_copy(hp_hbm.at[sidx.at[c1]], rows1, gs1).wait()
        pltpu.sync_copy(rows1, acc.at[didx.at[c1]], add=True)
        return _

    lax.fori_loop(0, NCHC // 2, body, None)


def _make_agg(d):
    @functools.partial(
        pl.kernel,
        out_type=jax.ShapeDtypeStruct((NC, NPAD, d), jnp.float32),
        mesh=_mesh,
        compiler_params=_sc_params,
        scratch_types=[
            pltpu.VMEM((NCHC, CHE), jnp.int32),
            pltpu.VMEM((NCHC, CHE), jnp.int32),
            pltpu.VMEM((CHE, d), jnp.float32),
            pltpu.VMEM((CHE, d), jnp.float32),
            pltpu.SemaphoreType.DMA,
            pltpu.SemaphoreType.DMA,
            pltpu.VMEM_SHARED((NPAD, d), jnp.float32),
        ],
    )
    def agg(hp_hbm, eixc_hbm, s_hbm, sidx, didx,
            rows0, rows1, gs0, gs1, acc):
        cid = lax.axis_index("c")
        sid = lax.axis_index("s")
        wid = cid * NS + sid
        pltpu.sync_copy(eixc_hbm.at[0, wid], sidx)
        pltpu.sync_copy(eixc_hbm.at[1, wid], didx)
        z = jnp.zeros((16,), jnp.float32)

        def zbody(r, _):
            for j in range(d // 16):
                rows0[r, pl.ds(j * 16, 16)] = z
            return _

        lax.fori_loop(0, 80, zbody, None)
        for q in range(SL // 80):
            pltpu.sync_copy(rows0.at[pl.ds(0, 80)],
                            acc.at[pl.ds(sid * SL + q * 80, 80)])
        plsc.subcore_barrier()
        _ring_body(hp_hbm, sidx, didx, rows0, rows1, gs0, gs1, acc)
        plsc.subcore_barrier()
        pltpu.sync_copy(acc.at[pl.ds(sid * SL, SL)],
                        s_hbm.at[cid, pl.ds(sid * SL, SL)])

    return agg


_agg1_call = _make_agg(D1P)
_agg2_call = _make_agg(D2)


def _mm1_body(cnt_ref, x_ref, w_ref, p_ref, dvc_ref, dv1_ref):
    i = pl.program_id(0)
    cb = cnt_ref[...]
    deg = cb[0:1, :] + cb[1:2, :] + 1.0
    col = lax.broadcasted_iota(jnp.int32, (1, BR), 1) + i * BR
    dvr = jnp.where(col < N, lax.rsqrt(deg), 0.0)
    dv1_ref[...] = dvr[0]
    dvc = jnp.transpose(dvr)
    dvc_ref[...] = dvc
    p_ref[...] = jnp.dot(x_ref[...], w_ref[...],
                         preferred_element_type=jnp.float32) * dvc


def _mm2_body(s_ref, p1_ref, dv_ref, b1_ref, w2_ref, out_ref):
    dv = dv_ref[...]
    h = dv * (s_ref[0] + s_ref[1] + p1_ref[...]) + b1_ref[...]
    h = jnp.maximum(h, 0.0)
    out_ref[...] = jnp.dot(h, w2_ref[...],
                           preferred_element_type=jnp.float32) * dv


def _fin_body(s_ref, p2_ref, dvc_ref, dv1_ref, g_ref, b2_ref,
              w3_ref, b3_ref, out_ref, acc_ref):
    i = pl.program_id(0)

    @pl.when(i == 0)
    def _z():
        acc_ref[...] = jnp.zeros_like(acc_ref)

    dvc = dvc_ref[...]
    h = jnp.maximum(
        dvc * (s_ref[0] + s_ref[1] + p2_ref[...]) + b2_ref[...], 0.0)
    gb = g_ref[...]
    dvr = dv1_ref[...][None, :]
    c = dvr * (gb[0:1, :] + gb[1:2, :] + dvr)
    acc_ref[...] += jnp.dot(c, h, preferred_element_type=jnp.float32)

    @pl.when(i == GRID - 1)
    def _f():
        out_ref[...] = jnp.dot(acc_ref[...] * (1.0 / N), w3_ref[...],
                               preferred_element_type=jnp.float32) + b3_ref[...]


def _col_spec(d):
    return pl.BlockSpec((BR, d), lambda i: (i, 0))


def _row_spec():
    return pl.BlockSpec((2, BR), lambda i: (0, i))


def _s_spec(d):
    return pl.BlockSpec((2, BR, d), lambda i: (0, i, 0))


def _const_spec(shape):
    return pl.BlockSpec(shape, lambda i: tuple(0 for _ in shape))


def kernel(x, edge_index, W1, b1, W2, b2, W3, b3):
    f32 = jnp.float32
    ei32 = edge_index.astype(jnp.int32)
    eixc = ei32.reshape(2, NT, NCHC, CHE)
    xpad = jnp.zeros((NPAD, D_IN), f32).at[:N].set(x)
    W1p = jnp.zeros((D_IN, D1P), f32).at[:, :D1].set(W1)
    b1p = jnp.zeros((1, D1P), f32).at[0, :D1].set(b1)
    W2p = jnp.zeros((D1P, D2), f32).at[:D1].set(W2)

    cnt2 = _deg_call(eixc)

    P1p, dinvc, dinv1 = pl.pallas_call(
        _mm1_body,
        grid=(GRID,),
        in_specs=[_row_spec(), _col_spec(D_IN), _const_spec((D_IN, D1P))],
        out_specs=[_col_spec(D1P), _col_spec(1),
                   pl.BlockSpec((BR,), lambda i: (i,))],
        out_shape=[jax.ShapeDtypeStruct((NPAD, D1P), f32),
                   jax.ShapeDtypeStruct((NPAD, 1), f32),
                   jax.ShapeDtypeStruct((NPAD,), f32)],
    )(cnt2, xpad, W1p)

    g2 = _g_call(dinv1, eixc)

    S1 = _agg1_call(P1p, eixc)

    P2p = pl.pallas_call(
        _mm2_body,
        grid=(GRID,),
        in_specs=[_s_spec(D1P), _col_spec(D1P), _col_spec(1),
                  _const_spec((1, D1P)), _const_spec((D1P, D2))],
        out_specs=_col_spec(D2),
        out_shape=jax.ShapeDtypeStruct((NPAD, D2), f32),
    )(S1, P1p, dinvc, b1p, W2p)

    S2 = _agg2_call(P2p, eixc)

    out = pl.pallas_call(
        _fin_body,
        grid=(GRID,),
        in_specs=[_s_spec(D2), _col_spec(D2), _col_spec(1),
                  pl.BlockSpec((BR,), lambda i: (i,)), _row_spec(),
                  _const_spec((1, D2)), _const_spec((D2, D3)),
                  _const_spec((1, D3))],
        out_specs=_const_spec((1, D3)),
        out_shape=jax.ShapeDtypeStruct((1, D3), f32),
        scratch_shapes=[pltpu.VMEM((1, D2), f32)],
    )(S2, P2p, dinvc, dinv1, g2, b2[None, :], W3, b3[None, :])

    return out
